# baseline (speedup 1.0000x reference)
import functools

import jax
import jax.numpy as jnp
from jax import lax
from jax.experimental import pallas as pl
from jax.experimental.pallas import tpu as pltpu
from jax.experimental.pallas import tpu_sc as plsc

BATCH = 4
SEQ = 4096
D = 2048
P = 10
L = 16
NW = 32
CH = 8

SC_BATCHES = 1
SC_ROWS = SC_BATCHES * SEQ
RPW = SC_ROWS // NW
NCH = RPW // CH
VPR = D // L


def _sc_body(x_hbm, emb_hbm, out_hbm, emb_v, bin_v, bout_v,
             sem_i0, sem_i1, sem_o0, sem_o1):
    wid = lax.axis_index("s") * 2 + lax.axis_index("c")
    base = wid * RPW
    ph0 = lax.rem(lax.rem(base, SEQ), P)

    sems_in = (sem_i0, sem_i1)
    sems_out = (sem_o0, sem_o1)

    def in_copy(c, slot):
        return pltpu.make_async_copy(
            x_hbm.at[pl.ds(base + c * CH, CH)], bin_v.at[slot],
            sems_in[slot])

    def out_copy(c, slot):
        return pltpu.make_async_copy(
            bout_v.at[slot], out_hbm.at[pl.ds(base + c * CH, CH)],
            sems_out[slot])

    def compute(c, slot):
        pv = [lax.rem(ph0 + c * CH + k, P) for k in range(CH)]

        def jbody(j, carry):
            off = pl.multiple_of(j * L, L)
            for k in range(CH):
                e = emb_v[pv[k], pl.ds(off, L)]
                bout_v[slot, k, pl.ds(off, L)] = bin_v[slot, k, pl.ds(off, L)] + e
            return carry

        lax.fori_loop(0, VPR, jbody, 0)

    pltpu.sync_copy(emb_hbm, emb_v)
    in_copy(0, 0).start()
    in_copy(1, 1).start()

    for c in (0, 1):
        slot = c & 1
        in_copy(c, slot).wait()
        compute(c, slot)
        out_copy(c, slot).start()
        in_copy(c + 2, slot).start()

    def chunk_pair(g, carry):
        for b in range(2):
            c = g * 2 + b
            in_copy(c, b).wait()
            out_copy(c - 2, b).wait()
            compute(c, b)
            out_copy(c, b).start()
            in_copy(c + 2, b).start()
        return carry

    lax.fori_loop(1, NCH // 2 - 1, chunk_pair, 0)

    for c in (NCH - 2, NCH - 1):
        slot = c & 1
        in_copy(c, slot).wait()
        out_copy(c - 2, slot).wait()
        compute(c, slot)
        out_copy(c, slot).start()
    out_copy(NCH - 2, 0).wait()
    out_copy(NCH - 1, 1).wait()


def _sc_part(x_rows, emb32):
    mesh = plsc.VectorSubcoreMesh(core_axis_name="c", subcore_axis_name="s")
    fn = functools.partial(
        pl.kernel,
        mesh=mesh,
        out_type=jax.ShapeDtypeStruct((SC_ROWS, D), jnp.float32),
        scratch_types=[
            pltpu.VMEM((P, D), jnp.float32),
            pltpu.VMEM((2, CH, D), jnp.float32),
            pltpu.VMEM((2, CH, D), jnp.float32),
            pltpu.SemaphoreType.DMA,
            pltpu.SemaphoreType.DMA,
            pltpu.SemaphoreType.DMA,
            pltpu.SemaphoreType.DMA,
        ],
    )(_sc_body)
    return fn(x_rows, emb32)


def _tc_block_kernel(x_ref, emb_ref, o_ref, *, seq_block: int):
    s0 = pl.program_id(1) * seq_block
    rows = jax.lax.broadcasted_iota(jnp.int32, (seq_block, P), 0) + s0
    phases = jax.lax.broadcasted_iota(jnp.int32, (seq_block, P), 1)
    one_hot = (rows % P == phases).astype(jnp.float32)
    emb_block = jnp.dot(one_hot, emb_ref[...],
                        preferred_element_type=jnp.float32)
    o_ref[...] = x_ref[...] + emb_block[None, :, :]


def _tc_part(x_tc, emb32):
    n_batch = x_tc.shape[0]
    seq_block = 512
    grid = (n_batch, SEQ // seq_block)
    fn = pl.pallas_call(
        functools.partial(_tc_block_kernel, seq_block=seq_block),
        grid=grid,
        in_specs=[
            pl.BlockSpec((1, seq_block, D), lambda b, s: (b, s, 0)),
            pl.BlockSpec((P, D), lambda b, s: (0, 0)),
        ],
        out_specs=pl.BlockSpec((1, seq_block, D), lambda b, s: (b, s, 0)),
        out_shape=jax.ShapeDtypeStruct(x_tc.shape, x_tc.dtype),
    )
    return fn(x_tc, emb32)


def kernel(x, embedding):
    emb32 = embedding.astype(jnp.float32)
    n_tc = BATCH - SC_BATCHES
    out_tc = _tc_part(x[:n_tc], emb32)
    out_sc = _sc_part(x[n_tc:].reshape(SC_ROWS, D), emb32)
    return jnp.concatenate(
        [out_tc, out_sc.reshape(SC_BATCHES, SEQ, D)], axis=0)
